# baseline (speedup 1.0000x reference)
import jax
import jax.numpy as jnp
from jax import lax
from jax.experimental import pallas as pl
from jax.experimental.pallas import tpu as pltpu
from jax.experimental.pallas import tpu_sc as plsc

_NC = 2
_NS = 16
_NW = _NC * _NS

_B = 16384
_CHUNK = 128
_BPW = _B // _NW
_NCHUNK = _BPW // _CHUNK


def _compact_body(gu_ref, gi_ref, ou_ref, oi_ref):
    ou_ref[...] = jnp.concatenate(
        [gu_ref[m::4, :] for m in range(4)], axis=1)
    oi_ref[...] = jnp.concatenate(
        [gi_ref[m::4, :] for m in range(4)], axis=1)


def _tc_compact(eu_gmf, ei_gmf):
    v, n_lat = eu_gmf.shape
    pack = 128 // n_lat
    blk = 4000
    grid = (v // blk,)
    return pl.pallas_call(
        _compact_body,
        grid=grid,
        in_specs=[
            pl.BlockSpec((blk, n_lat), lambda i: (i, 0)),
            pl.BlockSpec((blk, n_lat), lambda i: (i, 0)),
        ],
        out_specs=[
            pl.BlockSpec((blk // pack, 128), lambda i: (i, 0)),
            pl.BlockSpec((blk // pack, 128), lambda i: (i, 0)),
        ],
        out_shape=[
            jax.ShapeDtypeStruct((v // pack, 128), jnp.float32),
            jax.ShapeDtypeStruct((v // pack, 128), jnp.float32),
        ],
    )(eu_gmf, ei_gmf)


def _sc_mlp_body(user, item, mu, mi, um_out, im_out,
                 idx_u, idx_i, um_v, im_v, s0, s1):
    wid = lax.axis_index("s") * _NC + lax.axis_index("c")
    base = wid * _BPW
    for j in range(_NCHUNK):
        pltpu.sync_copy(user.at[pl.ds(base + j * _CHUNK, _CHUNK)], idx_u.at[j])
        pltpu.sync_copy(item.at[pl.ds(base + j * _CHUNK, _CHUNK)], idx_i.at[j])
    for j in range(_NCHUNK):
        row = base + j * _CHUNK
        c0 = pltpu.async_copy(mu.at[idx_u.at[j]], um_v, s0)
        c1 = pltpu.async_copy(mi.at[idx_i.at[j]], im_v, s1)
        c0.wait()
        pltpu.sync_copy(um_v, um_out.at[pl.ds(row, _CHUNK)])
        c1.wait()
        pltpu.sync_copy(im_v, im_out.at[pl.ds(row, _CHUNK)])


def _sc_mlp_gather(user, item, eu_mlp, ei_mlp):
    mlp_d = eu_mlp.shape[1]
    mesh = plsc.VectorSubcoreMesh(core_axis_name="c", subcore_axis_name="s",
                                  num_cores=_NC, num_subcores=_NS)
    f = pl.kernel(
        _sc_mlp_body,
        out_type=[
            jax.ShapeDtypeStruct((_B, mlp_d), jnp.float32),
            jax.ShapeDtypeStruct((_B, mlp_d), jnp.float32),
        ],
        mesh=mesh,
        scratch_types=[
            pltpu.VMEM((_NCHUNK, _CHUNK), jnp.int32),
            pltpu.VMEM((_NCHUNK, _CHUNK), jnp.int32),
            pltpu.VMEM((_CHUNK, mlp_d), jnp.float32),
            pltpu.VMEM((_CHUNK, mlp_d), jnp.float32),
            pltpu.SemaphoreType.DMA,
            pltpu.SemaphoreType.DMA,
        ],
    )
    return f(user, item, eu_mlp, ei_mlp)


def _sc_gmf_body(user, item, gu_c, gi_c, pk_out,
                 idx_u, idx_i, idx_uh, idx_ih, ugw_v, igw_v, pk_v, s2, s3):
    wid = lax.axis_index("s") * _NC + lax.axis_index("c")
    base = wid * _BPW
    n_lat = 32
    for j in range(_NCHUNK):
        pltpu.sync_copy(user.at[pl.ds(base + j * _CHUNK, _CHUNK)], idx_u.at[j])
        pltpu.sync_copy(item.at[pl.ds(base + j * _CHUNK, _CHUNK)], idx_i.at[j])
    for j in range(_NCHUNK):
        for k in range(_CHUNK // 16):
            s = pl.ds(k * 16, 16)
            idx_uh[j, s] = lax.shift_right_logical(idx_u[j, s], 2)
            idx_ih[j, s] = lax.shift_right_logical(idx_i[j, s], 2)
    for j in range(_NCHUNK):
        row = base + j * _CHUNK
        c2 = pltpu.async_copy(gu_c.at[idx_uh.at[j]], ugw_v, s2)
        c3 = pltpu.async_copy(gi_c.at[idx_ih.at[j]], igw_v, s3)
        c2.wait()
        c3.wait()

        def erow(g, _):
            vu = idx_u[j, pl.ds(g * 16, 16)]
            vi = idx_i[j, pl.ds(g * 16, 16)]
            for k in range(16):
                r = g * 16 + k
                ou = (vu[k] & 3) * n_lat
                pk_v[r, pl.ds(0, 16)] = ugw_v[r, pl.ds(ou, 16)]
                pk_v[r, pl.ds(16, 16)] = ugw_v[r, pl.ds(ou + 16, 16)]
                oi = (vi[k] & 3) * n_lat
                pk_v[r, pl.ds(32, 16)] = igw_v[r, pl.ds(oi, 16)]
                pk_v[r, pl.ds(48, 16)] = igw_v[r, pl.ds(oi + 16, 16)]
            return 0

        lax.fori_loop(0, _CHUNK // 16, erow, 0)
        pltpu.sync_copy(pk_v, pk_out.at[pl.ds(row, _CHUNK)])


def _sc_gmf_gather(user, item, gu_c, gi_c, n_lat):
    mesh = plsc.VectorSubcoreMesh(core_axis_name="c", subcore_axis_name="s",
                                  num_cores=_NC, num_subcores=_NS)
    f = pl.kernel(
        _sc_gmf_body,
        out_type=jax.ShapeDtypeStruct((_B, 128), jnp.float32),
        mesh=mesh,
        scratch_types=[
            pltpu.VMEM((_NCHUNK, _CHUNK), jnp.int32),
            pltpu.VMEM((_NCHUNK, _CHUNK), jnp.int32),
            pltpu.VMEM((_NCHUNK, _CHUNK), jnp.int32),
            pltpu.VMEM((_NCHUNK, _CHUNK), jnp.int32),
            pltpu.VMEM((_CHUNK, 128), jnp.float32),
            pltpu.VMEM((_CHUNK, 128), jnp.float32),
            pltpu.VMEM((_CHUNK, 128), jnp.float32),
            pltpu.SemaphoreType.DMA,
            pltpu.SemaphoreType.DMA,
        ],
    )
    return f(user, item, gu_c, gi_c)


def _tc_mlp_body(pk_ref, um_ref, im_ref,
                 w1u_ref, w1i_ref, b1_ref, w2_ref, b2_ref, w3_ref, b3_ref,
                 wp_ref, bp_ref, out_ref):
    n_lat = wp_ref.shape[0] // 2
    gmf = pk_ref[:, :n_lat] * pk_ref[:, n_lat:2 * n_lat]
    h = jnp.dot(um_ref[...], w1u_ref[...], preferred_element_type=jnp.float32)
    h += jnp.dot(im_ref[...], w1i_ref[...], preferred_element_type=jnp.float32)
    h = jax.nn.relu(h + b1_ref[...])
    h = jax.nn.relu(jnp.dot(h, w2_ref[...],
                            preferred_element_type=jnp.float32) + b2_ref[...])
    h = jax.nn.relu(jnp.dot(h, w3_ref[...],
                            preferred_element_type=jnp.float32) + b3_ref[...])
    wp = wp_ref[...]
    acc = jnp.dot(gmf, wp[:n_lat], preferred_element_type=jnp.float32)
    acc += jnp.dot(h, wp[n_lat:], preferred_element_type=jnp.float32)
    out_ref[...] = acc + bp_ref[0]


def _tc_mlp(pk, um, im, W1, b1, W2, b2, W3, b3, Wp, bp):
    mlp_d = um.shape[1]
    n_lat = mlp_d // 4
    blk = 2048
    grid = (_B // blk,)
    full = lambda shape: pl.BlockSpec(shape, lambda i: (0,) * len(shape))
    out = pl.pallas_call(
        _tc_mlp_body,
        grid=grid,
        in_specs=[
            pl.BlockSpec((blk, 128), lambda i: (i, 0)),
            pl.BlockSpec((blk, mlp_d), lambda i: (i, 0)),
            pl.BlockSpec((blk, mlp_d), lambda i: (i, 0)),
            full((mlp_d, mlp_d)),
            full((mlp_d, mlp_d)),
            full((1, mlp_d)),
            full((mlp_d, mlp_d // 2)),
            full((1, mlp_d // 2)),
            full((mlp_d // 2, n_lat)),
            full((1, n_lat)),
            full((2 * n_lat, 1)),
            full((1, 1)),
        ],
        out_specs=pl.BlockSpec((blk, 1), lambda i: (i, 0)),
        out_shape=jax.ShapeDtypeStruct((_B, 1), jnp.float32),
    )(pk, um, im,
      W1[:mlp_d], W1[mlp_d:], b1.reshape(1, -1),
      W2, b2.reshape(1, -1), W3, b3.reshape(1, -1),
      Wp, bp.reshape(1, 1))
    return out.reshape(-1)


def kernel(user, item, eu_gmf, ei_gmf, eu_mlp, ei_mlp,
           W1, b1, W2, b2, W3, b3, Wp, bp):
    user = user.astype(jnp.int32)
    item = item.astype(jnp.int32)
    um, im = _sc_mlp_gather(user, item, eu_mlp, ei_mlp)
    gu_c, gi_c = _tc_compact(eu_gmf, ei_gmf)
    pk = _sc_gmf_gather(user, item, gu_c, gi_c, eu_gmf.shape[1])
    return _tc_mlp(pk, um, im, W1, b1, W2, b2, W3, b3, Wp, bp)


# speedup vs baseline: 1.3670x; 1.3670x over previous
import jax
import jax.numpy as jnp
from jax import lax
from jax.experimental import pallas as pl
from jax.experimental.pallas import tpu as pltpu
from jax.experimental.pallas import tpu_sc as plsc

_NC = 2
_NS = 16
_NW = _NC * _NS

_B = 16384
_CHUNK = 128
_BPW = _B // _NW
_NCHUNK = _BPW // _CHUNK


def _sc_mlp_body(user, item, mu, mi, um_out, im_out,
                 idx_u, idx_i, um_v, im_v, s0, s1):
    wid = lax.axis_index("s") * _NC + lax.axis_index("c")
    base = wid * _BPW
    for j in range(_NCHUNK):
        pltpu.sync_copy(user.at[pl.ds(base + j * _CHUNK, _CHUNK)], idx_u.at[j])
        pltpu.sync_copy(item.at[pl.ds(base + j * _CHUNK, _CHUNK)], idx_i.at[j])
    for j in range(_NCHUNK):
        row = base + j * _CHUNK
        c0 = pltpu.async_copy(mu.at[idx_u.at[j]], um_v, s0)
        c1 = pltpu.async_copy(mi.at[idx_i.at[j]], im_v, s1)
        c0.wait()
        pltpu.sync_copy(um_v, um_out.at[pl.ds(row, _CHUNK)])
        c1.wait()
        pltpu.sync_copy(im_v, im_out.at[pl.ds(row, _CHUNK)])


def _sc_mlp_gather(user, item, eu_mlp, ei_mlp):
    mlp_d = eu_mlp.shape[1]
    mesh = plsc.VectorSubcoreMesh(core_axis_name="c", subcore_axis_name="s",
                                  num_cores=_NC, num_subcores=_NS)
    f = pl.kernel(
        _sc_mlp_body,
        out_type=[
            jax.ShapeDtypeStruct((_B, mlp_d), jnp.float32),
            jax.ShapeDtypeStruct((_B, mlp_d), jnp.float32),
        ],
        mesh=mesh,
        scratch_types=[
            pltpu.VMEM((_NCHUNK, _CHUNK), jnp.int32),
            pltpu.VMEM((_NCHUNK, _CHUNK), jnp.int32),
            pltpu.VMEM((_CHUNK, mlp_d), jnp.float32),
            pltpu.VMEM((_CHUNK, mlp_d), jnp.float32),
            pltpu.SemaphoreType.DMA,
            pltpu.SemaphoreType.DMA,
        ],
    )
    return f(user, item, eu_mlp, ei_mlp)


def _sc_gmf_body(user, item, gu, gi, pk_out,
                 idx_u, idx_i, ug_v, ig_v, pk_v, s2, s3):
    wid = lax.axis_index("s") * _NC + lax.axis_index("c")
    base = wid * _BPW
    n_lat = 32
    for j in range(_NCHUNK):
        pltpu.sync_copy(user.at[pl.ds(base + j * _CHUNK, _CHUNK)], idx_u.at[j])
        pltpu.sync_copy(item.at[pl.ds(base + j * _CHUNK, _CHUNK)], idx_i.at[j])
    for j in range(_NCHUNK):
        row = base + j * _CHUNK

        def row_body(g, _):
            vu = idx_u[j, pl.ds(g * 16, 16)]
            vi = idx_i[j, pl.ds(g * 16, 16)]
            for k in range(16):
                pltpu.async_copy(gu.at[vu[k]], ug_v.at[g * 16 + k], s2)
                pltpu.async_copy(gi.at[vi[k]], ig_v.at[g * 16 + k], s3)
            return 0

        lax.fori_loop(0, _CHUNK // 16, row_body, 0)
        pltpu.make_async_copy(gu.at[pl.ds(0, _CHUNK)], ug_v, s2).wait()
        pltpu.make_async_copy(gi.at[pl.ds(0, _CHUNK)], ig_v, s3).wait()

        def prow(g, _):
            for k in range(16):
                r = g * 16 + k
                pk_v[r, pl.ds(0, 16)] = ug_v[r, pl.ds(0, 16)]
                pk_v[r, pl.ds(16, 16)] = ug_v[r, pl.ds(16, 16)]
                pk_v[r, pl.ds(32, 16)] = ig_v[r, pl.ds(0, 16)]
                pk_v[r, pl.ds(48, 16)] = ig_v[r, pl.ds(16, 16)]
            return 0

        lax.fori_loop(0, _CHUNK // 16, prow, 0)
        pltpu.sync_copy(pk_v, pk_out.at[pl.ds(row, _CHUNK)])


def _sc_gmf_gather(user, item, gu, gi, n_lat):
    mesh = plsc.VectorSubcoreMesh(core_axis_name="c", subcore_axis_name="s",
                                  num_cores=_NC, num_subcores=_NS)
    f = pl.kernel(
        _sc_gmf_body,
        out_type=jax.ShapeDtypeStruct((_B, 128), jnp.float32),
        mesh=mesh,
        scratch_types=[
            pltpu.VMEM((_NCHUNK, _CHUNK), jnp.int32),
            pltpu.VMEM((_NCHUNK, _CHUNK), jnp.int32),
            pltpu.VMEM((_CHUNK, n_lat), jnp.float32),
            pltpu.VMEM((_CHUNK, n_lat), jnp.float32),
            pltpu.VMEM((_CHUNK, 128), jnp.float32),
            pltpu.SemaphoreType.DMA,
            pltpu.SemaphoreType.DMA,
        ],
    )
    return f(user, item, gu, gi)


def _tc_mlp_body(pk_ref, um_ref, im_ref,
                 w1u_ref, w1i_ref, b1_ref, w2_ref, b2_ref, w3_ref, b3_ref,
                 wp_ref, bp_ref, out_ref):
    n_lat = wp_ref.shape[0] // 2
    gmf = pk_ref[:, :n_lat] * pk_ref[:, n_lat:2 * n_lat]
    h = jnp.dot(um_ref[...], w1u_ref[...], preferred_element_type=jnp.float32)
    h += jnp.dot(im_ref[...], w1i_ref[...], preferred_element_type=jnp.float32)
    h = jax.nn.relu(h + b1_ref[...])
    h = jax.nn.relu(jnp.dot(h, w2_ref[...],
                            preferred_element_type=jnp.float32) + b2_ref[...])
    h = jax.nn.relu(jnp.dot(h, w3_ref[...],
                            preferred_element_type=jnp.float32) + b3_ref[...])
    wp = wp_ref[...]
    acc = jnp.dot(gmf, wp[:n_lat], preferred_element_type=jnp.float32)
    acc += jnp.dot(h, wp[n_lat:], preferred_element_type=jnp.float32)
    out_ref[...] = acc + bp_ref[0]


def _tc_mlp(pk, um, im, W1, b1, W2, b2, W3, b3, Wp, bp):
    mlp_d = um.shape[1]
    n_lat = mlp_d // 4
    blk = 2048
    grid = (_B // blk,)
    full = lambda shape: pl.BlockSpec(shape, lambda i: (0,) * len(shape))
    out = pl.pallas_call(
        _tc_mlp_body,
        grid=grid,
        in_specs=[
            pl.BlockSpec((blk, 128), lambda i: (i, 0)),
            pl.BlockSpec((blk, mlp_d), lambda i: (i, 0)),
            pl.BlockSpec((blk, mlp_d), lambda i: (i, 0)),
            full((mlp_d, mlp_d)),
            full((mlp_d, mlp_d)),
            full((1, mlp_d)),
            full((mlp_d, mlp_d // 2)),
            full((1, mlp_d // 2)),
            full((mlp_d // 2, n_lat)),
            full((1, n_lat)),
            full((2 * n_lat, 1)),
            full((1, 1)),
        ],
        out_specs=pl.BlockSpec((blk, 1), lambda i: (i, 0)),
        out_shape=jax.ShapeDtypeStruct((_B, 1), jnp.float32),
    )(pk, um, im,
      W1[:mlp_d], W1[mlp_d:], b1.reshape(1, -1),
      W2, b2.reshape(1, -1), W3, b3.reshape(1, -1),
      Wp, bp.reshape(1, 1))
    return out.reshape(-1)


def kernel(user, item, eu_gmf, ei_gmf, eu_mlp, ei_mlp,
           W1, b1, W2, b2, W3, b3, Wp, bp):
    user = user.astype(jnp.int32)
    item = item.astype(jnp.int32)
    um, im = _sc_mlp_gather(user, item, eu_mlp, ei_mlp)
    pk = _sc_gmf_gather(user, item, eu_gmf, ei_gmf, eu_gmf.shape[1])
    return _tc_mlp(pk, um, im, W1, b1, W2, b2, W3, b3, Wp, bp)


# speedup vs baseline: 1.5296x; 1.1189x over previous
import jax
import jax.numpy as jnp
from jax import lax
from jax.experimental import pallas as pl
from jax.experimental.pallas import tpu as pltpu
from jax.experimental.pallas import tpu_sc as plsc

_NC = 2
_NS = 16
_NW = _NC * _NS

_B = 16384
_CHUNK = 128
_BPW = _B // _NW
_NCHUNK = _BPW // _CHUNK


def _sc_mlp_body(user, item, mu, mi, um_out, im_out,
                 idx_u, idx_i, um_v, im_v, s0, s1):
    wid = lax.axis_index("s") * _NC + lax.axis_index("c")
    base = wid * _BPW
    for j in range(_NCHUNK):
        pltpu.sync_copy(user.at[pl.ds(base + j * _CHUNK, _CHUNK)], idx_u.at[j])
        pltpu.sync_copy(item.at[pl.ds(base + j * _CHUNK, _CHUNK)], idx_i.at[j])
    for j in range(_NCHUNK):
        row = base + j * _CHUNK
        c0 = pltpu.async_copy(mu.at[idx_u.at[j]], um_v, s0)
        c1 = pltpu.async_copy(mi.at[idx_i.at[j]], im_v, s1)
        c0.wait()
        pltpu.sync_copy(um_v, um_out.at[pl.ds(row, _CHUNK)])
        c1.wait()
        pltpu.sync_copy(im_v, im_out.at[pl.ds(row, _CHUNK)])


def _sc_mlp_gather(user, item, eu_mlp, ei_mlp):
    mlp_d = eu_mlp.shape[1]
    mesh = plsc.VectorSubcoreMesh(core_axis_name="c", subcore_axis_name="s",
                                  num_cores=_NC, num_subcores=_NS)
    f = pl.kernel(
        _sc_mlp_body,
        out_type=[
            jax.ShapeDtypeStruct((_B, mlp_d), jnp.float32),
            jax.ShapeDtypeStruct((_B, mlp_d), jnp.float32),
        ],
        mesh=mesh,
        scratch_types=[
            pltpu.VMEM((_NCHUNK, _CHUNK), jnp.int32),
            pltpu.VMEM((_NCHUNK, _CHUNK), jnp.int32),
            pltpu.VMEM((_CHUNK, mlp_d), jnp.float32),
            pltpu.VMEM((_CHUNK, mlp_d), jnp.float32),
            pltpu.SemaphoreType.DMA,
            pltpu.SemaphoreType.DMA,
        ],
    )
    return f(user, item, eu_mlp, ei_mlp)


def _sc_gmf_body(user, item, gu, gi, pk_out,
                 idx_u, idx_i, ug_v, ig_v, pk_v, s2, s3):
    wid = lax.axis_index("s") * _NC + lax.axis_index("c")
    base = wid * _BPW
    n_lat = 32
    for j in range(_NCHUNK):
        pltpu.sync_copy(user.at[pl.ds(base + j * _CHUNK, _CHUNK)], idx_u.at[j])
        pltpu.sync_copy(item.at[pl.ds(base + j * _CHUNK, _CHUNK)], idx_i.at[j])
    for j in range(_NCHUNK):
        row = base + j * _CHUNK

        def row_body(g, _):
            vu = idx_u[j, pl.ds(g * 16, 16)]
            vi = idx_i[j, pl.ds(g * 16, 16)]
            for k in range(16):
                pltpu.async_copy(gu.at[vu[k]], ug_v.at[g * 16 + k], s2)
                pltpu.async_copy(gi.at[vi[k]], ig_v.at[g * 16 + k], s3)
            return 0

        lax.fori_loop(0, _CHUNK // 16, row_body, 0)
        pltpu.make_async_copy(gu.at[pl.ds(0, _CHUNK)], ug_v, s2).wait()
        pltpu.make_async_copy(gi.at[pl.ds(0, _CHUNK)], ig_v, s3).wait()

        def prow(g, _):
            for k in range(16):
                r = g * 16 + k
                pk_v[r, pl.ds(0, 16)] = ug_v[r, pl.ds(0, 16)]
                pk_v[r, pl.ds(16, 16)] = ug_v[r, pl.ds(16, 16)]
                pk_v[r, pl.ds(32, 16)] = ig_v[r, pl.ds(0, 16)]
                pk_v[r, pl.ds(48, 16)] = ig_v[r, pl.ds(16, 16)]
            return 0

        lax.fori_loop(0, _CHUNK // 16, prow, 0)
        pltpu.sync_copy(pk_v, pk_out.at[pl.ds(row, _CHUNK)])


def _sc_gmf_gather(user, item, gu, gi, n_lat):
    mesh = plsc.VectorSubcoreMesh(core_axis_name="c", subcore_axis_name="s",
                                  num_cores=_NC, num_subcores=_NS)
    f = pl.kernel(
        _sc_gmf_body,
        out_type=jax.ShapeDtypeStruct((_B, 128), jnp.float32),
        mesh=mesh,
        scratch_types=[
            pltpu.VMEM((_NCHUNK, _CHUNK), jnp.int32),
            pltpu.VMEM((_NCHUNK, _CHUNK), jnp.int32),
            pltpu.VMEM((_CHUNK, n_lat), jnp.float32),
            pltpu.VMEM((_CHUNK, n_lat), jnp.float32),
            pltpu.VMEM((_CHUNK, 128), jnp.float32),
            pltpu.SemaphoreType.DMA,
            pltpu.SemaphoreType.DMA,
        ],
    )
    return f(user, item, gu, gi)


def _tc_h3_body(um_ref, im_ref, w1u_ref, w1i_ref, b1_ref,
                w2_ref, b2_ref, w3_ref, b3_ref, out_ref):
    h = jnp.dot(um_ref[...], w1u_ref[...], preferred_element_type=jnp.float32)
    h += jnp.dot(im_ref[...], w1i_ref[...], preferred_element_type=jnp.float32)
    h = jax.nn.relu(h + b1_ref[...])
    h = jax.nn.relu(jnp.dot(h, w2_ref[...],
                            preferred_element_type=jnp.float32) + b2_ref[...])
    h = jax.nn.relu(jnp.dot(h, w3_ref[...],
                            preferred_element_type=jnp.float32) + b3_ref[...])
    out_ref[...] = h


def _tc_h3(um, im, W1, b1, W2, b2, W3, b3):
    mlp_d = um.shape[1]
    n_lat = mlp_d // 4
    blk = 2048
    grid = (_B // blk,)
    full = lambda shape: pl.BlockSpec(shape, lambda i: (0,) * len(shape))
    return pl.pallas_call(
        _tc_h3_body,
        grid=grid,
        in_specs=[
            pl.BlockSpec((blk, mlp_d), lambda i: (i, 0)),
            pl.BlockSpec((blk, mlp_d), lambda i: (i, 0)),
            full((mlp_d, mlp_d)),
            full((mlp_d, mlp_d)),
            full((1, mlp_d)),
            full((mlp_d, mlp_d // 2)),
            full((1, mlp_d // 2)),
            full((mlp_d // 2, n_lat)),
            full((1, n_lat)),
        ],
        out_specs=pl.BlockSpec((blk, n_lat), lambda i: (i, 0)),
        out_shape=jax.ShapeDtypeStruct((_B, n_lat), jnp.float32),
    )(um, im, W1[:mlp_d], W1[mlp_d:], b1.reshape(1, -1),
      W2, b2.reshape(1, -1), W3, b3.reshape(1, -1))


def _tc_fuse_body(pk_ref, h_ref, wp_ref, bp_ref, out_ref):
    n_lat = wp_ref.shape[0] // 2
    gmf = pk_ref[:, :n_lat] * pk_ref[:, n_lat:2 * n_lat]
    wp = wp_ref[...]
    acc = jnp.dot(gmf, wp[:n_lat], preferred_element_type=jnp.float32)
    acc += jnp.dot(h_ref[...], wp[n_lat:], preferred_element_type=jnp.float32)
    out_ref[...] = acc + bp_ref[0]


def _tc_fuse(pk, h3, Wp, bp):
    n_lat = h3.shape[1]
    blk = 2048
    grid = (_B // blk,)
    full = lambda shape: pl.BlockSpec(shape, lambda i: (0,) * len(shape))
    out = pl.pallas_call(
        _tc_fuse_body,
        grid=grid,
        in_specs=[
            pl.BlockSpec((blk, 128), lambda i: (i, 0)),
            pl.BlockSpec((blk, n_lat), lambda i: (i, 0)),
            full((2 * n_lat, 1)),
            full((1, 1)),
        ],
        out_specs=pl.BlockSpec((blk, 1), lambda i: (i, 0)),
        out_shape=jax.ShapeDtypeStruct((_B, 1), jnp.float32),
    )(pk, h3, Wp, bp.reshape(1, 1))
    return out.reshape(-1)


def kernel(user, item, eu_gmf, ei_gmf, eu_mlp, ei_mlp,
           W1, b1, W2, b2, W3, b3, Wp, bp):
    user = user.astype(jnp.int32)
    item = item.astype(jnp.int32)
    um, im = _sc_mlp_gather(user, item, eu_mlp, ei_mlp)
    pk = _sc_gmf_gather(user, item, eu_gmf, ei_gmf, eu_gmf.shape[1])
    h3 = _tc_h3(um, im, W1, b1, W2, b2, W3, b3)
    return _tc_fuse(pk, h3, Wp, bp)
